# baseline (speedup 1.0000x reference)
import jax
import jax.numpy as jnp
from jax import lax
from jax.experimental import pallas as pl
from jax.experimental.pallas import tpu as pltpu
from jax.experimental.pallas import tpu_sc as plsc

NUM_ROWS = 1000000
DIM = 32
BATCH = 16384
SEQ = 50
NUM_TOKENS = BATCH * SEQ
NUM_CORES = 2
NUM_SUBCORES = 16
NUM_WORKERS = NUM_CORES * NUM_SUBCORES

N_FULL_BLK = NUM_ROWS // 128
REM_COLS = NUM_ROWS - N_FULL_BLK * 128
BLK_F32 = 128 * DIM

XJB = 7
XCH = 2048
XUNITS = XJB * (BATCH // XCH)


NKMAX = 246


def _fmt_body(
    w_t, x_t, w_tail, w_flat, idx_flat,
    win0, win1, wout0, wout1, xin_v, sem, osem, xsem,
):
    wid = lax.axis_index("s") * NUM_CORES + lax.axis_index("c")
    wins = (win0, win1)
    wouts = (wout0, wout1)

    iota16 = lax.iota(jnp.int32, 16)

    def start_in(c, b):
        pltpu.make_async_copy(
            w_t.at[:, pl.ds(c * 128, 128)], wins[b], sem
        ).start()

    def wait_in():
        pltpu.make_async_copy(w_t.at[:, pl.ds(0, 128)], win0, sem).wait()

    def start_out(c, b):
        pltpu.make_async_copy(
            wouts[b], w_flat.at[pl.ds(c * BLK_F32, BLK_F32)], osem
        ).start()

    def wait_out():
        pltpu.make_async_copy(wout0, w_flat.at[pl.ds(0, BLK_F32)], osem).wait()

    ibase = iota16 * DIM

    def shuffle(b, ngroups):
        for f in range(DIM):
            for g in range(ngroups):
                vals = wins[b][f, pl.ds(g * 16, 16)]
                plsc.store_scatter(
                    wouts[b], [ibase + (g * 16 * DIM + f)], vals
                )

    nk = (N_FULL_BLK - wid + NUM_WORKERS - 1) // NUM_WORKERS

    @pl.when(nk > 0)
    def _():
        start_in(wid, 0)

    @pl.when(nk > 1)
    def _():
        start_in(NUM_WORKERS + wid, 1)

    @pl.loop(0, NKMAX, step=2)
    def _ko(ko):
        for b in range(2):
            k = ko + b

            @pl.when(k < nk)
            def _():
                wait_in()

                @pl.when(k >= 2)
                def _():
                    wait_out()

                shuffle(b, 8)
                start_out(k * NUM_WORKERS + wid, b)

                @pl.when(k + 2 < nk)
                def _():
                    start_in((k + 2) * NUM_WORKERS + wid, b)

    @pl.when(nk >= 2)
    def _():
        wait_out()

    @pl.when(nk >= 1)
    def _():
        wait_out()

    @pl.when(wid == 4)
    def _():
        pltpu.sync_copy(w_tail, wout0.at[pl.ds(0, REM_COLS * DIM)])
        pltpu.sync_copy(
            wout0.at[pl.ds(0, REM_COLS * DIM)],
            w_flat.at[pl.ds(N_FULL_BLK * BLK_F32, REM_COLS * DIM)],
        )

    @pl.loop(0, XUNITS)
    def _u(u):
        @pl.when(u % NUM_WORKERS == wid)
        def _():
            jb = u // (BATCH // XCH)
            ic = u % (BATCH // XCH)
            nrow = jnp.where(jb == XJB - 1, SEQ - 8 * (XJB - 1), 8)
            pltpu.sync_copy(
                x_t.at[pl.ds(jb * 8, 8), pl.ds(ic * XCH, XCH)], xin_v
            )

            @pl.loop(0, nrow)
            def _r(r):
                pltpu.make_async_copy(
                    xin_v.at[r],
                    idx_flat.at[pl.ds((jb * 8 + r) * BATCH + ic * XCH, XCH)],
                    xsem,
                ).start()
                pltpu.make_async_copy(
                    xin_v.at[r],
                    idx_flat.at[pl.ds(0, XCH)],
                    xsem,
                ).wait()


B_PER_W = NUM_TOKENS // NUM_WORKERS
NBUF = 4
CHUNK = 640
N_CHUNKS = B_PER_W // CHUNK


def _gather_body(idx_hbm, table_hbm, out_hbm, idx_v, rows_v, gsem, wsem):
    wid = lax.axis_index("s") * NUM_CORES + lax.axis_index("c")
    base = wid * B_PER_W
    pltpu.sync_copy(idx_hbm.at[pl.ds(base, B_PER_W)], idx_v)

    def start_gather(g, b):
        pltpu.make_async_copy(
            table_hbm.at[idx_v.at[pl.ds(g * CHUNK, CHUNK)]],
            rows_v.at[b],
            gsem,
        ).start()

    def wait_gather(b):
        pltpu.make_async_copy(
            table_hbm.at[idx_v.at[pl.ds(0, CHUNK)]], rows_v.at[b], gsem
        ).wait()

    def start_wb(g, b):
        pltpu.make_async_copy(
            rows_v.at[b], out_hbm.at[pl.ds(base + g * CHUNK, CHUNK)], wsem
        ).start()

    def wait_wb(b):
        pltpu.make_async_copy(
            rows_v.at[b], out_hbm.at[pl.ds(base, CHUNK)], wsem
        ).wait()

    for b in range(NBUF):
        start_gather(b, b)

    @pl.loop(0, N_CHUNKS, step=NBUF)
    def _round(go):
        for b in range(NBUF):
            g = go + b
            wait_gather(b)
            start_wb(g, b)

            @pl.when(g + NBUF < N_CHUNKS)
            def _():
                wait_wb(b)
                start_gather(g + NBUF, b)

    for b in range(NBUF):
        wait_wb(b)


@jax.jit
def _embed(x, weight):
    mesh = plsc.VectorSubcoreMesh(core_axis_name="c", subcore_axis_name="s")
    w_flat, idx_flat = pl.kernel(
        _fmt_body,
        out_type=(
            jax.ShapeDtypeStruct((NUM_ROWS * DIM,), jnp.float32),
            jax.ShapeDtypeStruct((NUM_TOKENS,), jnp.int32),
        ),
        mesh=mesh,
        scratch_types=[
            pltpu.VMEM((DIM, 128), jnp.float32),
            pltpu.VMEM((DIM, 128), jnp.float32),
            pltpu.VMEM((BLK_F32,), jnp.float32),
            pltpu.VMEM((BLK_F32,), jnp.float32),
            pltpu.VMEM((8, XCH), jnp.int32),
            pltpu.SemaphoreType.DMA,
            pltpu.SemaphoreType.DMA,
            pltpu.SemaphoreType.DMA,
        ],
        compiler_params=pltpu.CompilerParams(needs_layout_passes=False),
    )(weight.T, x.T, weight[N_FULL_BLK * 128 :].reshape(-1))

    table = w_flat.reshape(NUM_ROWS, DIM)
    out2 = pl.kernel(
        _gather_body,
        out_type=jax.ShapeDtypeStruct((NUM_TOKENS, DIM), jnp.float32),
        mesh=mesh,
        scratch_types=[
            pltpu.VMEM((B_PER_W,), jnp.int32),
            pltpu.VMEM((NBUF, CHUNK, DIM), jnp.float32),
            pltpu.SemaphoreType.DMA,
            pltpu.SemaphoreType.DMA,
        ],
        compiler_params=pltpu.CompilerParams(use_tc_tiling_on_sc=False),
    )(idx_flat, table)

    return out2.reshape(SEQ, BATCH, DIM).transpose(1, 0, 2)


def kernel(x, weight):
    return _embed(x, weight)


# speedup vs baseline: 1.4970x; 1.4970x over previous
import jax
import jax.numpy as jnp
from jax import lax
from jax.experimental import pallas as pl
from jax.experimental.pallas import tpu as pltpu
from jax.experimental.pallas import tpu_sc as plsc

NUM_ROWS = 1000000
DIM = 32
BATCH = 16384
SEQ = 50
NUM_TOKENS = BATCH * SEQ
NUM_CORES = 2
NUM_SUBCORES = 16
NUM_WORKERS = NUM_CORES * NUM_SUBCORES

N_FULL_BLK = NUM_ROWS // 128
REM_COLS = NUM_ROWS - N_FULL_BLK * 128
BLK_F32 = 128 * DIM

XJB = 7
XCH = 2048
XUNITS = XJB * (BATCH // XCH)


NKMAX = 246


def _fmt_body(
    w_t, x_t, w_tail, w_flat, idx_flat,
    win0, win1, wout0, wout1, xin_v, sem, osem, xsem,
):
    wid = lax.axis_index("s") * NUM_CORES + lax.axis_index("c")
    wins = (win0, win1)
    wouts = (wout0, wout1)

    iota16 = lax.iota(jnp.int32, 16)

    def start_in(c, b):
        pltpu.make_async_copy(
            w_t.at[:, pl.ds(c * 128, 128)], wins[b], sem
        ).start()

    def wait_in():
        pltpu.make_async_copy(w_t.at[:, pl.ds(0, 128)], win0, sem).wait()

    def start_out(c, b):
        pltpu.make_async_copy(
            wouts[b], w_flat.at[pl.ds(c * BLK_F32, BLK_F32)], osem
        ).start()

    def wait_out():
        pltpu.make_async_copy(wout0, w_flat.at[pl.ds(0, BLK_F32)], osem).wait()

    rot = [(iota16 + s) % 16 for s in range(16)]
    rot32 = [r * DIM + iota16 for r in rot]

    def shuffle(b, ngroups):
        @pl.loop(0, ngroups)
        def _g(g):
            il0 = g * 16
            for f0 in (0, 16):
                rowv = iota16 + f0
                for s in range(16):
                    vals = plsc.load_gather(wins[b], [rowv, il0 + rot[s]])
                    plsc.store_scatter(
                        wouts[b], [rot32[s] + (il0 * DIM + f0)], vals
                    )

    nk = (N_FULL_BLK - wid + NUM_WORKERS - 1) // NUM_WORKERS

    @pl.when(nk > 0)
    def _():
        start_in(wid, 0)

    @pl.when(nk > 1)
    def _():
        start_in(NUM_WORKERS + wid, 1)

    @pl.loop(0, NKMAX, step=2)
    def _ko(ko):
        for b in range(2):
            k = ko + b

            @pl.when(k < nk)
            def _():
                wait_in()

                @pl.when(k >= 2)
                def _():
                    wait_out()

                shuffle(b, 8)
                start_out(k * NUM_WORKERS + wid, b)

                @pl.when(k + 2 < nk)
                def _():
                    start_in((k + 2) * NUM_WORKERS + wid, b)

    @pl.when(nk >= 2)
    def _():
        wait_out()

    @pl.when(nk >= 1)
    def _():
        wait_out()

    @pl.when(wid == 4)
    def _():
        pltpu.sync_copy(w_tail, wout0.at[pl.ds(0, REM_COLS * DIM)])
        pltpu.sync_copy(
            wout0.at[pl.ds(0, REM_COLS * DIM)],
            w_flat.at[pl.ds(N_FULL_BLK * BLK_F32, REM_COLS * DIM)],
        )

    @pl.loop(0, XUNITS)
    def _u(u):
        @pl.when(u % NUM_WORKERS == wid)
        def _():
            jb = u // (BATCH // XCH)
            ic = u % (BATCH // XCH)
            nrow = jnp.where(jb == XJB - 1, SEQ - 8 * (XJB - 1), 8)
            pltpu.sync_copy(
                x_t.at[pl.ds(jb * 8, 8), pl.ds(ic * XCH, XCH)], xin_v
            )

            @pl.loop(0, nrow)
            def _r(r):
                pltpu.make_async_copy(
                    xin_v.at[r],
                    idx_flat.at[pl.ds((jb * 8 + r) * BATCH + ic * XCH, XCH)],
                    xsem,
                ).start()
                pltpu.make_async_copy(
                    xin_v.at[r],
                    idx_flat.at[pl.ds(0, XCH)],
                    xsem,
                ).wait()


B_PER_W = NUM_TOKENS // NUM_WORKERS
NBUF = 4
CHUNK = 640
N_CHUNKS = B_PER_W // CHUNK


def _gather_body(idx_hbm, table_hbm, out_hbm, idx_v, rows_v, gsem, wsem):
    wid = lax.axis_index("s") * NUM_CORES + lax.axis_index("c")
    base = wid * B_PER_W
    pltpu.sync_copy(idx_hbm.at[pl.ds(base, B_PER_W)], idx_v)

    def start_gather(g, b):
        pltpu.make_async_copy(
            table_hbm.at[idx_v.at[pl.ds(g * CHUNK, CHUNK)]],
            rows_v.at[b],
            gsem,
        ).start()

    def wait_gather(b):
        pltpu.make_async_copy(
            table_hbm.at[idx_v.at[pl.ds(0, CHUNK)]], rows_v.at[b], gsem
        ).wait()

    def start_wb(g, b):
        pltpu.make_async_copy(
            rows_v.at[b], out_hbm.at[pl.ds(base + g * CHUNK, CHUNK)], wsem
        ).start()

    def wait_wb(b):
        pltpu.make_async_copy(
            rows_v.at[b], out_hbm.at[pl.ds(base, CHUNK)], wsem
        ).wait()

    for b in range(NBUF):
        start_gather(b, b)

    @pl.loop(0, N_CHUNKS, step=NBUF)
    def _round(go):
        for b in range(NBUF):
            g = go + b
            wait_gather(b)
            start_wb(g, b)

            @pl.when(g + NBUF < N_CHUNKS)
            def _():
                wait_wb(b)
                start_gather(g + NBUF, b)

    for b in range(NBUF):
        wait_wb(b)


@jax.jit
def _embed(x, weight):
    mesh = plsc.VectorSubcoreMesh(core_axis_name="c", subcore_axis_name="s")
    w_flat, idx_flat = pl.kernel(
        _fmt_body,
        out_type=(
            jax.ShapeDtypeStruct((NUM_ROWS * DIM,), jnp.float32),
            jax.ShapeDtypeStruct((NUM_TOKENS,), jnp.int32),
        ),
        mesh=mesh,
        scratch_types=[
            pltpu.VMEM((DIM, 128), jnp.float32),
            pltpu.VMEM((DIM, 128), jnp.float32),
            pltpu.VMEM((BLK_F32,), jnp.float32),
            pltpu.VMEM((BLK_F32,), jnp.float32),
            pltpu.VMEM((8, XCH), jnp.int32),
            pltpu.SemaphoreType.DMA,
            pltpu.SemaphoreType.DMA,
            pltpu.SemaphoreType.DMA,
        ],
        compiler_params=pltpu.CompilerParams(needs_layout_passes=False),
    )(weight.T, x.T, weight[N_FULL_BLK * 128 :].reshape(-1))

    table = w_flat.reshape(NUM_ROWS, DIM)
    out2 = pl.kernel(
        _gather_body,
        out_type=jax.ShapeDtypeStruct((NUM_TOKENS, DIM), jnp.float32),
        mesh=mesh,
        scratch_types=[
            pltpu.VMEM((B_PER_W,), jnp.int32),
            pltpu.VMEM((NBUF, CHUNK, DIM), jnp.float32),
            pltpu.SemaphoreType.DMA,
            pltpu.SemaphoreType.DMA,
        ],
        compiler_params=pltpu.CompilerParams(use_tc_tiling_on_sc=False),
    )(idx_flat, table)

    return out2.reshape(SEQ, BATCH, DIM).transpose(1, 0, 2)


def kernel(x, weight):
    return _embed(x, weight)
